# baseline (speedup 1.0000x reference)
import functools

import jax
import jax.numpy as jnp
from jax import lax
from jax.experimental import pallas as pl
from jax.experimental.pallas import tpu as pltpu
from jax.experimental.pallas import tpu_sc as plsc

NC = 2
NS = 16
L = 16
NW = NC * NS

B = 256
D = 4 * 64 * 64
R = B // NW
UNROLL = 8


def _body(x_hbm, ts_hbm, n_hbm, sa_hbm, so_hbm, out_hbm,
          sa_v, so_v, ts_v, xb0, xb1, nb0, nb1, ob0, ob1,
          in_sem0, in_sem1, out_sem0, out_sem1):
    c = lax.axis_index("c")
    s = lax.axis_index("s")
    wid = s * NC + c
    base = wid * R

    xbufs = (xb0, xb1)
    nbufs = (nb0, nb1)
    obufs = (ob0, ob1)
    in_sems = (in_sem0, in_sem1)
    out_sems = (out_sem0, out_sem1)

    pltpu.sync_copy(sa_hbm, sa_v)
    pltpu.sync_copy(so_hbm, so_v)
    pltpu.sync_copy(ts_hbm, ts_v)

    pltpu.async_copy(x_hbm.at[base], xbufs[0], in_sems[0])
    pltpu.async_copy(n_hbm.at[base], nbufs[0], in_sems[0])

    for j in range(R):
        slot = j % 2
        nxt = (j + 1) % 2
        if j + 1 < R:
            pltpu.async_copy(x_hbm.at[base + j + 1], xbufs[nxt], in_sems[nxt])
            pltpu.async_copy(n_hbm.at[base + j + 1], nbufs[nxt], in_sems[nxt])

        jv = jnp.full((L,), j, jnp.int32) + base
        tv = plsc.load_gather(ts_v, [jv])
        sab = plsc.load_gather(sa_v, [tv])
        sob = plsc.load_gather(so_v, [tv])

        pltpu.make_async_copy(x_hbm.at[base + j], xbufs[slot], in_sems[slot]).wait()
        pltpu.make_async_copy(n_hbm.at[base + j], nbufs[slot], in_sems[slot]).wait()
        if j >= 2:
            pltpu.make_async_copy(obufs[slot], out_hbm.at[base + j - 2],
                                  out_sems[slot]).wait()

        xs = xbufs[slot]
        ns = nbufs[slot]
        os_ = obufs[slot]

        def step(i, _):
            for u in range(UNROLL):
                off = (i * UNROLL + u) * L
                xv = xs[pl.ds(off, L)]
                nv = ns[pl.ds(off, L)]
                os_[pl.ds(off, L)] = sab * xv + sob * nv
            return 0

        lax.fori_loop(0, D // (L * UNROLL), step, 0)

        pltpu.async_copy(obufs[slot], out_hbm.at[base + j], out_sems[slot])

    pltpu.make_async_copy(obufs[R % 2], out_hbm.at[base + R - 2],
                          out_sems[R % 2]).wait()
    pltpu.make_async_copy(obufs[(R - 1) % 2], out_hbm.at[base + R - 1],
                          out_sems[(R - 1) % 2]).wait()


@jax.jit
def kernel(x_start, timesteps, noise, sqrt_alpha_cumprod,
           sqrt_one_minus_alpha_cumprod):
    x2 = x_start.reshape(B, D)
    n2 = noise.reshape(B, D)
    sa_p = jnp.zeros((1024,), jnp.float32).at[:1000].set(sqrt_alpha_cumprod)
    so_p = jnp.zeros((1024,), jnp.float32).at[:1000].set(
        sqrt_one_minus_alpha_cumprod)

    k = functools.partial(
        pl.kernel,
        out_type=jax.ShapeDtypeStruct((B, D), jnp.float32),
        mesh=plsc.VectorSubcoreMesh(core_axis_name="c", subcore_axis_name="s"),
        compiler_params=pltpu.CompilerParams(needs_layout_passes=False),
        scratch_types=[
            pltpu.VMEM((1024,), jnp.float32),
            pltpu.VMEM((1024,), jnp.float32),
            pltpu.VMEM((B,), jnp.int32),
            pltpu.VMEM((D,), jnp.float32),
            pltpu.VMEM((D,), jnp.float32),
            pltpu.VMEM((D,), jnp.float32),
            pltpu.VMEM((D,), jnp.float32),
            pltpu.VMEM((D,), jnp.float32),
            pltpu.VMEM((D,), jnp.float32),
            pltpu.SemaphoreType.DMA,
            pltpu.SemaphoreType.DMA,
            pltpu.SemaphoreType.DMA,
            pltpu.SemaphoreType.DMA,
        ],
    )(_body)

    out = k(x2, timesteps, n2, sa_p, so_p)
    return out.reshape(x_start.shape)
